# baseline (speedup 1.0000x reference)
import jax
import jax.numpy as jnp
from jax.experimental import pallas as pl
from jax.experimental.pallas import tpu as pltpu

def _nms_kernel(x_ref, w_ref, b_ref, bx_ref, logits_ref, preds_ref,
                dT, tilemax, packidx, comm_ref):
    CP, N = dT.shape
    T = CP // 8
    C = bx_ref.shape[0]
    logits = jax.lax.dot_general(
        x_ref[...], w_ref[...],
        dimension_numbers=(((1,), (1,)), ((), ())),
        preferred_element_type=jnp.float32,
    )
    logits = logits + b_ref[...]
    logits_ref[...] = logits
    probs = jax.nn.softmax(logits, axis=1)
    lane_c = jax.lax.broadcasted_iota(jnp.int32, (1, C), 1)
    probs = jnp.where(lane_c == 0, -0.0, probs)
    dT[0:C, :] = probs.T
    dT[C:CP, :] = jnp.full((CP - C, N), -jnp.inf, jnp.float32)
    for t in range(T):
        tilemax[t:t + 1, :] = jnp.max(dT[t * 8:(t + 1) * 8, :], axis=0,
                                      keepdims=True)
    packidx[...] = (jax.lax.broadcasted_iota(jnp.int32, (T, N), 1) * 16
                    + jax.lax.broadcasted_iota(jnp.int32, (T, N), 0))
    lid = jax.lax.broadcasted_iota(jnp.int32, (1, N), 1)
    sub8 = jax.lax.broadcasted_iota(jnp.int32, (8, N), 0)

    preds_ref[...] = jnp.zeros((1, N), jnp.int32)
    comm_ref[...] = jnp.zeros((1, N), jnp.int32)

    def body(i, carry):
        comm = comm_ref[...]
        committed = comm != 0
        tm = tilemax[...]
        m = jnp.max(tm)
        pk = jnp.min(jnp.where(tm == m, packidx[...], jnp.int32(2 ** 30)))
        box = pk // 16
        t = pk - box * 16
        base = pl.multiple_of(t * 8, 8)
        slab = dT[pl.ds(base, 8), :]
        truez = ((slab == 0.0)
                 & (jax.lax.bitcast_convert_type(slab, jnp.int32) >= 0))
        cond_comm = jnp.logical_xor(truez, m < -0.5)
        cond_unc = slab == m
        is_comm = jnp.sum(jnp.where(lid == box, comm, 0)) > 0
        cond = (((cond_comm & is_comm) | (cond_unc & jnp.logical_not(is_comm)))
                & (lid == box) & (sub8 < C - base))
        s = jnp.min(jnp.where(cond, sub8, jnp.int32(2 ** 30)))
        cls = base + s
        selm = lid == box
        preds_ref[...] = jnp.where(selm, cls, preds_ref[...])

        @pl.when(is_comm)
        def _():
            d2 = dT[...]
            tz2 = ((d2 == 0.0)
                   & (jax.lax.bitcast_convert_type(d2, jnp.int32) >= 0))
            lane_f = jax.lax.broadcasted_iota(jnp.int32, (CP, N), 1)
            dT[...] = jnp.where(tz2 & (lane_f == box), -0.0, d2)
        sl = bx_ref[pl.ds(cls, 1), :, :][0]
        x1 = sl[0:1]
        y1 = sl[1:2]
        x2 = sl[2:3]
        y2 = sl[3:4]
        sx1 = jnp.sum(jnp.where(selm, x1, 0.0))
        sy1 = jnp.sum(jnp.where(selm, y1, 0.0))
        sx2 = jnp.sum(jnp.where(selm, x2, 0.0))
        sy2 = jnp.sum(jnp.where(selm, y2, 0.0))
        iw = jnp.maximum(jnp.minimum(x2, sx2) - jnp.maximum(x1, sx1) + 1.0, 0.0)
        ih = jnp.maximum(jnp.minimum(y2, sy2) - jnp.maximum(y1, sy1) + 1.0, 0.0)
        inters = iw * ih
        area = (x2 - x1 + 1.0) * (y2 - y1 + 1.0)
        sarea = (sx2 - sx1 + 1.0) * (sy2 - sy1 + 1.0)
        union = area + sarea - inters
        mask = (inters / union) >= 0.5
        row = dT[pl.ds(cls, 1), :]
        zero_w = jnp.where(committed & jnp.logical_not(selm), 0.0, -0.0)
        dT[pl.ds(cls, 1), :] = jnp.where(mask, zero_w, row)
        tcb = pl.multiple_of((cls // 8) * 8, 8)
        slab2 = dT[pl.ds(tcb, 8), :]
        slabmax = jnp.max(slab2, axis=0, keepdims=True)
        tm_row = tilemax[pl.ds(cls // 8, 1), :]
        tm_new = jnp.where(committed & mask, 0.0,
                           jnp.where(committed, tm_row, slabmax))
        tilemax[pl.ds(cls // 8, 1), :] = tm_new
        tm2 = tilemax[...]
        lane_t = jax.lax.broadcasted_iota(jnp.int32, (T, N), 1)
        tilemax[...] = jnp.where(lane_t == box, -1.0, tm2)
        comm_ref[...] = jnp.where(selm, 1, comm)
        return carry

    jax.lax.fori_loop(0, N, body, 0)


def kernel(x, boxes_per_cls, W, b):
    N, D = x.shape
    C = W.shape[0]
    CP = ((C + 7) // 8) * 8
    T = CP // 8
    boxesT = jnp.transpose(boxes_per_cls, (1, 2, 0))
    b2 = b.reshape(1, C)
    logits, preds = pl.pallas_call(
        _nms_kernel,
        out_shape=(
            jax.ShapeDtypeStruct((N, C), jnp.float32),
            jax.ShapeDtypeStruct((1, N), jnp.int32),
        ),
        scratch_shapes=[
            pltpu.VMEM((CP, N), jnp.float32),
            pltpu.VMEM((T, N), jnp.float32),
            pltpu.VMEM((T, N), jnp.int32),
            pltpu.VMEM((1, N), jnp.int32),
        ],
    )(x, W, b2, boxesT)
    return logits, preds.reshape(N)


# speedup vs baseline: 1.1453x; 1.1453x over previous
import jax
import jax.numpy as jnp
from jax.experimental import pallas as pl
from jax.experimental.pallas import tpu as pltpu


def _nms_kernel(x_ref, w_ref, b_ref, bx_ref, logits_ref, preds_ref,
                dT, best_ref, bcls_ref, comm_ref):
    CP, N = dT.shape
    C = bx_ref.shape[0]
    logits = jax.lax.dot_general(
        x_ref[...], w_ref[...],
        dimension_numbers=(((1,), (1,)), ((), ())),
        preferred_element_type=jnp.float32,
    )
    logits = logits + b_ref[...]
    logits_ref[...] = logits
    probs = jax.nn.softmax(logits, axis=1)
    lane_c = jax.lax.broadcasted_iota(jnp.int32, (1, C), 1)
    probs = jnp.where(lane_c == 0, -0.0, probs)
    dT[0:C, :] = probs.T
    if CP > C:
        dT[C:CP, :] = jnp.full((CP - C, N), -jnp.inf, jnp.float32)
    lid = jax.lax.broadcasted_iota(jnp.int32, (1, N), 1)
    subf = jax.lax.broadcasted_iota(jnp.int32, (CP, N), 0)

    d0 = dT[...]
    cm0 = jnp.max(d0, axis=0, keepdims=True)
    best_ref[...] = cm0
    bcls_ref[...] = jnp.min(jnp.where(d0 == cm0, subf, jnp.int32(C)),
                            axis=0, keepdims=True)
    preds_ref[...] = jnp.zeros((1, N), jnp.int32)
    comm_ref[...] = jnp.zeros((1, N), jnp.int32)

    def body(i, carry):
        best = best_ref[...]
        bcls = bcls_ref[...]
        comm = comm_ref[...]
        committed = comm != 0
        m = jnp.max(best)
        box = jnp.min(jnp.where(best == m, lid, jnp.int32(2 ** 30)))
        selm = lid == box
        all_neg = m < -0.5
        cls = jnp.where(all_neg, 0,
                        jnp.sum(jnp.where(selm, bcls, 0)))
        is_comm = jnp.sum(jnp.where(selm, comm, 0)) > 0
        preds_ref[...] = jnp.where(selm, cls, preds_ref[...])

        @pl.when(is_comm)
        def _():
            d2 = dT[...]
            tz2 = ((d2 == 0.0)
                   & (jax.lax.bitcast_convert_type(d2, jnp.int32) >= 0))
            lane_f = jax.lax.broadcasted_iota(jnp.int32, (CP, N), 1)
            dT[...] = jnp.where(tz2 & (lane_f == box), -0.0, d2)

        sl = bx_ref[pl.ds(cls, 1), :, :][0]
        x1 = sl[0:1]
        y1 = sl[1:2]
        x2 = sl[2:3]
        y2 = sl[3:4]
        sx1 = jnp.sum(jnp.where(selm, x1, 0.0))
        sy1 = jnp.sum(jnp.where(selm, y1, 0.0))
        sx2 = jnp.sum(jnp.where(selm, x2, 0.0))
        sy2 = jnp.sum(jnp.where(selm, y2, 0.0))
        iw = jnp.maximum(jnp.minimum(x2, sx2) - jnp.maximum(x1, sx1) + 1.0, 0.0)
        ih = jnp.maximum(jnp.minimum(y2, sy2) - jnp.maximum(y1, sy1) + 1.0, 0.0)
        inters = iw * ih
        area = (x2 - x1 + 1.0) * (y2 - y1 + 1.0)
        sarea = (sx2 - sx1 + 1.0) * (sy2 - sy1 + 1.0)
        union = area + sarea - inters
        mask = (inters / union) >= 0.5
        cbm = committed & jnp.logical_not(selm)
        row = dT[pl.ds(cls, 1), :]
        zero_w = jnp.where(cbm, 0.0, -0.0)
        dT[pl.ds(cls, 1), :] = jnp.where(mask, zero_w, row)
        hit = cbm & mask
        bcls = jnp.where(hit & ((best < -0.5) | (cls < bcls)), cls, bcls)
        best = jnp.where(hit, 0.0, best)
        best = jnp.where(selm, -1.0, best)
        bcls = jnp.where(selm, C, bcls)
        best_ref[...] = best
        bcls_ref[...] = bcls
        comm_ref[...] = jnp.where(selm, 1, comm)
        aff = (jnp.logical_not(committed) & jnp.logical_not(selm)
               & mask & (bcls_ref[...] == cls))
        any_aff = jnp.sum(jnp.where(aff, 1, 0)) > 0

        @pl.when(any_aff)
        def _():
            dr = dT[...]
            cm = jnp.max(dr, axis=0, keepdims=True)
            ca = jnp.min(jnp.where(dr == cm, subf, jnp.int32(C)),
                         axis=0, keepdims=True)
            best_ref[...] = jnp.where(aff, cm, best_ref[...])
            bcls_ref[...] = jnp.where(aff, ca, bcls_ref[...])

        return carry

    jax.lax.fori_loop(0, N, body, 0)


def kernel(x, boxes_per_cls, W, b):
    N, D = x.shape
    C = W.shape[0]
    CP = ((C + 7) // 8) * 8
    boxesT = jnp.transpose(boxes_per_cls, (1, 2, 0))
    b2 = b.reshape(1, C)
    logits, preds = pl.pallas_call(
        _nms_kernel,
        out_shape=(
            jax.ShapeDtypeStruct((N, C), jnp.float32),
            jax.ShapeDtypeStruct((1, N), jnp.int32),
        ),
        scratch_shapes=[
            pltpu.VMEM((CP, N), jnp.float32),
            pltpu.VMEM((1, N), jnp.float32),
            pltpu.VMEM((1, N), jnp.int32),
            pltpu.VMEM((1, N), jnp.int32),
        ],
    )(x, W, b2, boxesT)
    return logits, preds.reshape(N)


# speedup vs baseline: 1.2408x; 1.0834x over previous
import functools

import jax
import jax.numpy as jnp
from jax.experimental import pallas as pl
from jax.experimental.pallas import tpu as pltpu


def _dense_kernel(x_ref, w_ref, b_ref, logits_ref, dists_ref):
    C = w_ref.shape[0]
    N = x_ref.shape[0]
    NP = dists_ref.shape[1]
    logits = jax.lax.dot_general(
        x_ref[...], w_ref[...],
        dimension_numbers=(((1,), (1,)), ((), ())),
        preferred_element_type=jnp.float32,
    )
    logits = logits + b_ref[...]
    logits_ref[...] = logits
    probs = jax.nn.softmax(logits, axis=1)
    lane_c = jax.lax.broadcasted_iota(jnp.int32, (1, C), 1)
    probs = jnp.where(lane_c == 0, -0.0, probs)
    dists_ref[:, 0:N] = probs.T
    dists_ref[:, N:NP] = jnp.full((C, NP - N), -jnp.inf, jnp.float32)


def _nms_kernel(d_ref, bx_ref, preds_ref, dT, best_ref, bcls_ref, comm_ref,
                *, n_steps):
    C = d_ref.shape[0]
    CP = dT.shape[0]
    dT[0:C] = d_ref[...]
    if CP > C:
        dT[C:CP] = jnp.full((CP - C, 8, 128), -jnp.inf, jnp.float32)
    idx2 = (jax.lax.broadcasted_iota(jnp.int32, (8, 128), 0) * 128
            + jax.lax.broadcasted_iota(jnp.int32, (8, 128), 1))
    subf = jax.lax.broadcasted_iota(jnp.int32, (CP, 8, 128), 0)

    d0 = dT[...]
    cm0 = jnp.max(d0, axis=0)
    best_ref[...] = cm0
    bcls_ref[...] = jnp.min(jnp.where(d0 == cm0, subf, jnp.int32(C)), axis=0)
    preds_ref[...] = jnp.zeros((8, 128), jnp.int32)
    comm_ref[...] = jnp.where(cm0 == -jnp.inf, 1, 0)

    def body(i, carry):
        best = best_ref[...]
        bcls = bcls_ref[...]
        comm = comm_ref[...]
        committed = comm != 0
        m = jnp.max(best)
        box = jnp.min(jnp.where(best == m, idx2, jnp.int32(2 ** 30)))
        selm = idx2 == box
        all_neg = m < -0.5
        cls = jnp.where(all_neg, 0, jnp.sum(jnp.where(selm, bcls, 0)))
        is_comm = jnp.sum(jnp.where(selm, comm, 0)) > 0
        preds_ref[...] = jnp.where(selm, cls, preds_ref[...])

        @pl.when(is_comm)
        def _():
            d2 = dT[...]
            tz2 = ((d2 == 0.0)
                   & (jax.lax.bitcast_convert_type(d2, jnp.int32) >= 0))
            dT[...] = jnp.where(tz2 & selm, -0.0, d2)

        slb = bx_ref[pl.ds(cls, 1), :, :, :][0]
        x1 = slb[0]
        y1 = slb[1]
        x2 = slb[2]
        y2 = slb[3]
        sx1 = jnp.sum(jnp.where(selm, x1, 0.0))
        sy1 = jnp.sum(jnp.where(selm, y1, 0.0))
        sx2 = jnp.sum(jnp.where(selm, x2, 0.0))
        sy2 = jnp.sum(jnp.where(selm, y2, 0.0))
        iw = jnp.maximum(jnp.minimum(x2, sx2) - jnp.maximum(x1, sx1) + 1.0, 0.0)
        ih = jnp.maximum(jnp.minimum(y2, sy2) - jnp.maximum(y1, sy1) + 1.0, 0.0)
        inters = iw * ih
        area = (x2 - x1 + 1.0) * (y2 - y1 + 1.0)
        sarea = (sx2 - sx1 + 1.0) * (sy2 - sy1 + 1.0)
        union = area + sarea - inters
        mask = (inters / union) >= 0.5
        cbm = committed & jnp.logical_not(selm)
        row = dT[pl.ds(cls, 1), :, :][0]
        zero_w = jnp.where(cbm, 0.0, -0.0)
        dT[pl.ds(cls, 1), :, :] = jnp.where(mask, zero_w, row)[None]
        hit = cbm & mask
        bcls = jnp.where(hit & ((best < -0.5) | (cls < bcls)), cls, bcls)
        best = jnp.where(hit, 0.0, best)
        best = jnp.where(selm, -1.0, best)
        bcls = jnp.where(selm, C, bcls)
        best_ref[...] = best
        bcls_ref[...] = bcls
        comm_ref[...] = jnp.where(selm, 1, comm)
        aff = (jnp.logical_not(committed) & jnp.logical_not(selm)
               & mask & (bcls == cls))
        any_aff = jnp.sum(jnp.where(aff, 1, 0)) > 0

        @pl.when(any_aff)
        def _():
            dr = dT[...]
            cm = jnp.max(dr, axis=0)
            ca = jnp.min(jnp.where(dr == cm, subf, jnp.int32(C)), axis=0)
            best_ref[...] = jnp.where(aff, cm, best_ref[...])
            bcls_ref[...] = jnp.where(aff, ca, bcls_ref[...])

        return carry

    jax.lax.fori_loop(0, n_steps, body, 0)


def kernel(x, boxes_per_cls, W, b):
    N, D = x.shape
    C = W.shape[0]
    CP = ((C + 7) // 8) * 8
    NP = 1024
    b2 = b.reshape(1, C)
    logits, dists = pl.pallas_call(
        _dense_kernel,
        out_shape=(
            jax.ShapeDtypeStruct((N, C), jnp.float32),
            jax.ShapeDtypeStruct((C, NP), jnp.float32),
        ),
    )(x, W, b2)
    dists2 = dists.reshape(C, 8, 128)
    boxesT = jnp.transpose(boxes_per_cls, (1, 2, 0))
    boxesP = jnp.concatenate(
        [boxesT, jnp.zeros((C, 4, NP - N), jnp.float32)], axis=2
    ).reshape(C, 4, 8, 128)
    preds = pl.pallas_call(
        functools.partial(_nms_kernel, n_steps=N),
        out_shape=jax.ShapeDtypeStruct((8, 128), jnp.int32),
        scratch_shapes=[
            pltpu.VMEM((CP, 8, 128), jnp.float32),
            pltpu.VMEM((8, 128), jnp.float32),
            pltpu.VMEM((8, 128), jnp.int32),
            pltpu.VMEM((8, 128), jnp.int32),
        ],
    )(dists2, boxesP)
    return logits, preds.reshape(NP)[:N]


# speedup vs baseline: 1.4576x; 1.1748x over previous
import functools

import jax
import jax.numpy as jnp
from jax.experimental import pallas as pl
from jax.experimental.pallas import tpu as pltpu


def _dense_kernel(x_ref, w_ref, b_ref, logits_ref, dists_ref):
    C = w_ref.shape[0]
    N = x_ref.shape[0]
    NP = dists_ref.shape[1]
    logits = jax.lax.dot_general(
        x_ref[...], w_ref[...],
        dimension_numbers=(((1,), (1,)), ((), ())),
        preferred_element_type=jnp.float32,
    )
    logits = logits + b_ref[...]
    logits_ref[...] = logits
    probs = jax.nn.softmax(logits, axis=1)
    lane_c = jax.lax.broadcasted_iota(jnp.int32, (1, C), 1)
    probs = jnp.where(lane_c == 0, -0.0, probs)
    dists_ref[:, 0:N] = probs.T
    dists_ref[:, N:NP] = jnp.full((C, NP - N), -jnp.inf, jnp.float32)


def _nms_kernel(d_ref, bx_ref, preds_ref, dT, best_ref, bcls_ref, comm_ref,
                *, n_steps):
    C = d_ref.shape[0]
    CP = dT.shape[0]
    dT[0:C] = d_ref[...]
    if CP > C:
        dT[C:CP] = jnp.full((CP - C, 8, 128), -jnp.inf, jnp.float32)
    idx2 = (jax.lax.broadcasted_iota(jnp.int32, (8, 128), 0) * 128
            + jax.lax.broadcasted_iota(jnp.int32, (8, 128), 1))
    subf = jax.lax.broadcasted_iota(jnp.int32, (CP, 8, 128), 0)

    d0 = dT[...]
    cm0 = jnp.max(d0, axis=0)
    best_ref[...] = cm0
    bcls_ref[...] = jnp.min(jnp.where(d0 == cm0, subf, jnp.int32(C)), axis=0)
    preds_ref[...] = jnp.zeros((8, 128), jnp.int32)
    comm_ref[...] = jnp.where(cm0 == -jnp.inf, 1, 0)

    def body(i, carry):
        best = best_ref[...]
        bcls = bcls_ref[...]
        comm = comm_ref[...]
        committed = comm != 0
        m = jnp.max(best)
        pack = jnp.min(jnp.where(best == m,
                                 idx2 * 256 + bcls * 2 + comm,
                                 jnp.int32(2 ** 30)))
        box = pack // 256
        selm = idx2 == box
        all_neg = m < -0.5
        cls = jnp.where(all_neg, 0, (pack // 2) % 128)
        is_comm = (pack % 2) > 0
        preds_ref[...] = jnp.where(selm, cls, preds_ref[...])

        @pl.when(is_comm)
        def _():
            d2 = dT[...]
            tz2 = ((d2 == 0.0)
                   & (jax.lax.bitcast_convert_type(d2, jnp.int32) >= 0))
            dT[...] = jnp.where(tz2 & selm, -0.0, d2)

        slb = bx_ref[pl.ds(cls, 1), :, :, :][0]
        x1 = slb[0]
        y1 = slb[1]
        x2 = slb[2]
        y2 = slb[3]
        sx1 = jnp.sum(jnp.where(selm, x1, 0.0))
        sy1 = jnp.sum(jnp.where(selm, y1, 0.0))
        sx2 = jnp.sum(jnp.where(selm, x2, 0.0))
        sy2 = jnp.sum(jnp.where(selm, y2, 0.0))
        iw = jnp.maximum(jnp.minimum(x2, sx2) - jnp.maximum(x1, sx1) + 1.0, 0.0)
        ih = jnp.maximum(jnp.minimum(y2, sy2) - jnp.maximum(y1, sy1) + 1.0, 0.0)
        inters = iw * ih
        area = (x2 - x1 + 1.0) * (y2 - y1 + 1.0)
        sarea = (sx2 - sx1 + 1.0) * (sy2 - sy1 + 1.0)
        union = area + sarea - inters
        mask = (inters / union) >= 0.5
        cbm = committed & jnp.logical_not(selm)
        row = dT[pl.ds(cls, 1), :, :][0]
        zero_w = jnp.where(cbm, 0.0, -0.0)
        dT[pl.ds(cls, 1), :, :] = jnp.where(mask, zero_w, row)[None]
        hit = cbm & mask
        bcls = jnp.where(hit & ((best < -0.5) | (cls < bcls)), cls, bcls)
        best = jnp.where(hit, 0.0, best)
        best = jnp.where(selm, -1.0, best)
        bcls = jnp.where(selm, C, bcls)
        best_ref[...] = best
        bcls_ref[...] = bcls
        comm_ref[...] = jnp.where(selm, 1, comm)
        aff = (jnp.logical_not(committed) & jnp.logical_not(selm)
               & mask & (bcls == cls))
        dr = dT[...]
        cm = jnp.max(dr, axis=0)
        ca = jnp.min(jnp.where(dr == cm, subf, jnp.int32(C)), axis=0)
        best_ref[...] = jnp.where(aff, cm, best_ref[...])
        bcls_ref[...] = jnp.where(aff, ca, bcls_ref[...])

        return carry

    jax.lax.fori_loop(0, n_steps, body, 0)


def kernel(x, boxes_per_cls, W, b):
    N, D = x.shape
    C = W.shape[0]
    CP = ((C + 7) // 8) * 8
    NP = 1024
    b2 = b.reshape(1, C)
    logits, dists = pl.pallas_call(
        _dense_kernel,
        out_shape=(
            jax.ShapeDtypeStruct((N, C), jnp.float32),
            jax.ShapeDtypeStruct((C, NP), jnp.float32),
        ),
    )(x, W, b2)
    dists2 = dists.reshape(C, 8, 128)
    boxesT = jnp.transpose(boxes_per_cls, (1, 2, 0))
    boxesP = jnp.concatenate(
        [boxesT, jnp.zeros((C, 4, NP - N), jnp.float32)], axis=2
    ).reshape(C, 4, 8, 128)
    preds = pl.pallas_call(
        functools.partial(_nms_kernel, n_steps=N),
        out_shape=jax.ShapeDtypeStruct((8, 128), jnp.int32),
        scratch_shapes=[
            pltpu.VMEM((CP, 8, 128), jnp.float32),
            pltpu.VMEM((8, 128), jnp.float32),
            pltpu.VMEM((8, 128), jnp.int32),
            pltpu.VMEM((8, 128), jnp.int32),
        ],
    )(dists2, boxesP)
    return logits, preds.reshape(NP)[:N]


# speedup vs baseline: 1.7403x; 1.1939x over previous
import functools

import jax
import jax.numpy as jnp
from jax.experimental import pallas as pl
from jax.experimental.pallas import tpu as pltpu


def _dense_kernel(x_ref, w_ref, b_ref, logits_ref, dists_ref):
    C = w_ref.shape[0]
    N = x_ref.shape[0]
    NP = dists_ref.shape[1]
    logits = jax.lax.dot_general(
        x_ref[...], w_ref[...],
        dimension_numbers=(((1,), (1,)), ((), ())),
        preferred_element_type=jnp.float32,
    )
    logits = logits + b_ref[...]
    logits_ref[...] = logits
    probs = jax.nn.softmax(logits, axis=1)
    lane_c = jax.lax.broadcasted_iota(jnp.int32, (1, C), 1)
    probs = jnp.where(lane_c == 0, -0.0, probs)
    dists_ref[:, 0:N] = probs.T
    dists_ref[:, N:NP] = jnp.full((C, NP - N), -jnp.inf, jnp.float32)


def _nms_kernel(d_ref, bx_ref, preds_ref, dT, best_ref, bcls_ref, comm_ref,
                cbx_ref, cst_ref, sc_ref, *, n_steps):
    C = d_ref.shape[0]
    CP = dT.shape[0]
    dT[0:C] = d_ref[...]
    if CP > C:
        dT[C:CP] = jnp.full((CP - C, 8, 128), -jnp.inf, jnp.float32)
    idx2 = (jax.lax.broadcasted_iota(jnp.int32, (8, 128), 0) * 128
            + jax.lax.broadcasted_iota(jnp.int32, (8, 128), 1))
    subf = jax.lax.broadcasted_iota(jnp.int32, (CP, 8, 128), 0)

    d0 = dT[...]
    cm0 = jnp.max(d0, axis=0)
    best_ref[...] = cm0
    bc0 = jnp.min(jnp.where(d0 == cm0, subf, jnp.int32(C)), axis=0)
    bcls_ref[...] = bc0
    preds_ref[...] = jnp.zeros((8, 128), jnp.int32)
    comm_ref[...] = jnp.where(cm0 == -jnp.inf, 1, 0)
    for k in range(4):
        acc = jnp.zeros((8, 128), jnp.float32)
        for c in range(C):
            acc = jnp.where(bc0 == c, bx_ref[c, k], acc)
        cbx_ref[k] = acc
    cst_ref[...] = jnp.zeros((8, 128), jnp.int32)

    def body(i, carry):
        best = best_ref[...]
        bcls = bcls_ref[...]
        comm = comm_ref[...]
        committed = comm != 0
        cstale = cst_ref[...]
        m = jnp.max(best)
        cand = best == m
        pack = jnp.min(jnp.where(cand,
                                 idx2 * 512 + bcls * 4 + comm * 2 + cstale,
                                 jnp.int32(2 ** 30)))
        tmax = jnp.max(jnp.where(cand, idx2, -1))
        c0 = jnp.sum(jnp.where(cand, cbx_ref[0], 0.0))
        c1 = jnp.sum(jnp.where(cand, cbx_ref[1], 0.0))
        c2 = jnp.sum(jnp.where(cand, cbx_ref[2], 0.0))
        c3 = jnp.sum(jnp.where(cand, cbx_ref[3], 0.0))
        box = pack // 512
        selm = idx2 == box
        all_neg = m < -0.5
        cls = jnp.where(all_neg, 0, (pack // 4) % 128)
        is_comm = ((pack // 2) % 2) > 0
        slow = (tmax != box) | all_neg | ((pack % 2) > 0)
        sc_ref[0] = jnp.full((8, 128), c0)
        sc_ref[1] = jnp.full((8, 128), c1)
        sc_ref[2] = jnp.full((8, 128), c2)
        sc_ref[3] = jnp.full((8, 128), c3)
        preds_ref[...] = jnp.where(selm, cls, preds_ref[...])

        @pl.when(is_comm)
        def _():
            d2 = dT[...]
            tz2 = ((d2 == 0.0)
                   & (jax.lax.bitcast_convert_type(d2, jnp.int32) >= 0))
            dT[...] = jnp.where(tz2 & selm, -0.0, d2)

        slb = bx_ref[pl.ds(cls, 1), :, :, :][0]
        x1 = slb[0]
        y1 = slb[1]
        x2 = slb[2]
        y2 = slb[3]

        @pl.when(slow)
        def _():
            sc_ref[0] = jnp.full((8, 128), jnp.sum(jnp.where(selm, x1, 0.0)))
            sc_ref[1] = jnp.full((8, 128), jnp.sum(jnp.where(selm, y1, 0.0)))
            sc_ref[2] = jnp.full((8, 128), jnp.sum(jnp.where(selm, x2, 0.0)))
            sc_ref[3] = jnp.full((8, 128), jnp.sum(jnp.where(selm, y2, 0.0)))
            cbx_ref[0] = jnp.where(selm, x1, cbx_ref[0])
            cbx_ref[1] = jnp.where(selm, y1, cbx_ref[1])
            cbx_ref[2] = jnp.where(selm, x2, cbx_ref[2])
            cbx_ref[3] = jnp.where(selm, y2, cbx_ref[3])
            cst_ref[...] = jnp.where(selm, 0, cst_ref[...])

        sx1 = sc_ref[0]
        sy1 = sc_ref[1]
        sx2 = sc_ref[2]
        sy2 = sc_ref[3]
        iw = jnp.maximum(jnp.minimum(x2, sx2) - jnp.maximum(x1, sx1) + 1.0, 0.0)
        ih = jnp.maximum(jnp.minimum(y2, sy2) - jnp.maximum(y1, sy1) + 1.0, 0.0)
        inters = iw * ih
        area = (x2 - x1 + 1.0) * (y2 - y1 + 1.0)
        sarea = (sx2 - sx1 + 1.0) * (sy2 - sy1 + 1.0)
        union = area + sarea - inters
        mask = (inters / union) >= 0.5
        cbm = committed & jnp.logical_not(selm)
        row = dT[pl.ds(cls, 1), :, :][0]
        zero_w = jnp.where(cbm, 0.0, -0.0)
        dT[pl.ds(cls, 1), :, :] = jnp.where(mask, zero_w, row)[None]
        hit = cbm & mask
        updc = hit & ((best < -0.5) | (cls < bcls))
        bcls = jnp.where(updc, cls, bcls)
        best = jnp.where(hit, 0.0, best)
        cbx_ref[0] = jnp.where(updc, x1, cbx_ref[0])
        cbx_ref[1] = jnp.where(updc, y1, cbx_ref[1])
        cbx_ref[2] = jnp.where(updc, x2, cbx_ref[2])
        cbx_ref[3] = jnp.where(updc, y2, cbx_ref[3])
        best = jnp.where(selm, -1.0, best)
        bcls = jnp.where(selm, C, bcls)
        best_ref[...] = best
        bcls_ref[...] = bcls
        comm_ref[...] = jnp.where(selm, 1, comm)
        aff = (jnp.logical_not(committed) & jnp.logical_not(selm)
               & mask & (bcls == cls))
        dr = dT[...]
        cm = jnp.max(dr, axis=0)
        ca = jnp.min(jnp.where(dr == cm, subf, jnp.int32(C)), axis=0)
        best_ref[...] = jnp.where(aff, cm, best_ref[...])
        bcls_ref[...] = jnp.where(aff, ca, bcls_ref[...])
        cst_ref[...] = jnp.where(aff, 1, cst_ref[...])

        return carry

    jax.lax.fori_loop(0, n_steps, body, 0)


def kernel(x, boxes_per_cls, W, b):
    N, D = x.shape
    C = W.shape[0]
    CP = ((C + 7) // 8) * 8
    NP = 1024
    b2 = b.reshape(1, C)
    logits, dists = pl.pallas_call(
        _dense_kernel,
        out_shape=(
            jax.ShapeDtypeStruct((N, C), jnp.float32),
            jax.ShapeDtypeStruct((C, NP), jnp.float32),
        ),
    )(x, W, b2)
    dists2 = dists.reshape(C, 8, 128)
    boxesT = jnp.transpose(boxes_per_cls, (1, 2, 0))
    boxesP = jnp.concatenate(
        [boxesT, jnp.zeros((C, 4, NP - N), jnp.float32)], axis=2
    ).reshape(C, 4, 8, 128)
    preds = pl.pallas_call(
        functools.partial(_nms_kernel, n_steps=N),
        out_shape=jax.ShapeDtypeStruct((8, 128), jnp.int32),
        scratch_shapes=[
            pltpu.VMEM((CP, 8, 128), jnp.float32),
            pltpu.VMEM((8, 128), jnp.float32),
            pltpu.VMEM((8, 128), jnp.int32),
            pltpu.VMEM((8, 128), jnp.int32),
            pltpu.VMEM((4, 8, 128), jnp.float32),
            pltpu.VMEM((8, 128), jnp.int32),
            pltpu.VMEM((4, 8, 128), jnp.float32),
        ],
    )(dists2, boxesP)
    return logits, preds.reshape(NP)[:N]


# speedup vs baseline: 1.7604x; 1.0116x over previous
import functools

import jax
import jax.numpy as jnp
from jax.experimental import pallas as pl
from jax.experimental.pallas import tpu as pltpu


def _dense_kernel(x_ref, w_ref, b_ref, logits_ref, dists_ref):
    C = w_ref.shape[0]
    N = x_ref.shape[0]
    NP = dists_ref.shape[1]
    logits = jax.lax.dot_general(
        x_ref[...], w_ref[...],
        dimension_numbers=(((1,), (1,)), ((), ())),
        preferred_element_type=jnp.float32,
    )
    logits = logits + b_ref[...]
    logits_ref[...] = logits
    probs = jax.nn.softmax(logits, axis=1)
    lane_c = jax.lax.broadcasted_iota(jnp.int32, (1, C), 1)
    probs = jnp.where(lane_c == 0, -0.0, probs)
    dists_ref[:, 0:N] = probs.T
    dists_ref[:, N:NP] = jnp.full((C, NP - N), -jnp.inf, jnp.float32)


def _nms_kernel(d_ref, bx_ref, preds_ref, dT, best_ref, bcls_ref, comm_ref,
                cbx_ref, cst_ref, sc_ref, *, n_steps):
    C = d_ref.shape[0]
    CP = dT.shape[0]
    dT[0:C] = d_ref[...]
    if CP > C:
        dT[C:CP] = jnp.full((CP - C, 8, 128), -jnp.inf, jnp.float32)
    idx2 = (jax.lax.broadcasted_iota(jnp.int32, (8, 128), 0) * 128
            + jax.lax.broadcasted_iota(jnp.int32, (8, 128), 1))
    subf = jax.lax.broadcasted_iota(jnp.int32, (CP, 8, 128), 0)

    d0 = dT[...]
    cm0 = jnp.max(d0, axis=0)
    best_ref[...] = cm0
    bc0 = jnp.min(jnp.where(d0 == cm0, subf, jnp.int32(C)), axis=0)
    bcls_ref[...] = bc0
    preds_ref[...] = jnp.zeros((8, 128), jnp.int32)
    comm_ref[...] = jnp.where(cm0 == -jnp.inf, 1, 0)
    for k in range(4):
        acc = jnp.zeros((8, 128), jnp.float32)
        for c in range(C):
            acc = jnp.where(bc0 == c, bx_ref[c, k], acc)
        cbx_ref[k] = acc
    cst_ref[...] = jnp.zeros((8, 128), jnp.int32)

    def body(i, carry):
        dr_pre = dT[...]
        best = best_ref[...]
        bcls = bcls_ref[...]
        comm = comm_ref[...]
        committed = comm != 0
        cstale = cst_ref[...]
        m = jnp.max(best)
        cand = best == m
        pack = jnp.min(jnp.where(cand,
                                 idx2 * 512 + bcls * 4 + comm * 2 + cstale,
                                 jnp.int32(2 ** 30)))
        tmax = jnp.max(jnp.where(cand, idx2, -1))
        c0 = jnp.sum(jnp.where(cand, cbx_ref[0], 0.0))
        c1 = jnp.sum(jnp.where(cand, cbx_ref[1], 0.0))
        c2 = jnp.sum(jnp.where(cand, cbx_ref[2], 0.0))
        c3 = jnp.sum(jnp.where(cand, cbx_ref[3], 0.0))
        box = pack // 512
        selm = idx2 == box
        all_neg = m < -0.5
        cls = jnp.where(all_neg, 0, (pack // 4) % 128)
        is_comm = ((pack // 2) % 2) > 0
        slow = (tmax != box) | all_neg | ((pack % 2) > 0)
        sc_ref[0] = jnp.full((8, 128), c0)
        sc_ref[1] = jnp.full((8, 128), c1)
        sc_ref[2] = jnp.full((8, 128), c2)
        sc_ref[3] = jnp.full((8, 128), c3)
        preds_ref[...] = jnp.where(selm, cls, preds_ref[...])

        @pl.when(is_comm)
        def _():
            d2 = dT[...]
            tz2 = ((d2 == 0.0)
                   & (jax.lax.bitcast_convert_type(d2, jnp.int32) >= 0))
            dT[...] = jnp.where(tz2 & selm, -0.0, d2)

        slb = bx_ref[pl.ds(cls, 1), :, :, :][0]
        x1 = slb[0]
        y1 = slb[1]
        x2 = slb[2]
        y2 = slb[3]

        @pl.when(slow)
        def _():
            sc_ref[0] = jnp.full((8, 128), jnp.sum(jnp.where(selm, x1, 0.0)))
            sc_ref[1] = jnp.full((8, 128), jnp.sum(jnp.where(selm, y1, 0.0)))
            sc_ref[2] = jnp.full((8, 128), jnp.sum(jnp.where(selm, x2, 0.0)))
            sc_ref[3] = jnp.full((8, 128), jnp.sum(jnp.where(selm, y2, 0.0)))
            cbx_ref[0] = jnp.where(selm, x1, cbx_ref[0])
            cbx_ref[1] = jnp.where(selm, y1, cbx_ref[1])
            cbx_ref[2] = jnp.where(selm, x2, cbx_ref[2])
            cbx_ref[3] = jnp.where(selm, y2, cbx_ref[3])
            cst_ref[...] = jnp.where(selm, 0, cst_ref[...])

        sx1 = sc_ref[0]
        sy1 = sc_ref[1]
        sx2 = sc_ref[2]
        sy2 = sc_ref[3]
        iw = jnp.maximum(jnp.minimum(x2, sx2) - jnp.maximum(x1, sx1) + 1.0, 0.0)
        ih = jnp.maximum(jnp.minimum(y2, sy2) - jnp.maximum(y1, sy1) + 1.0, 0.0)
        inters = iw * ih
        area = (x2 - x1 + 1.0) * (y2 - y1 + 1.0)
        sarea = (sx2 - sx1 + 1.0) * (sy2 - sy1 + 1.0)
        union = area + sarea - inters
        mask = (inters / union) >= 0.5
        cbm = committed & jnp.logical_not(selm)
        row = dT[pl.ds(cls, 1), :, :][0]
        zero_w = jnp.where(cbm, 0.0, -0.0)
        dT[pl.ds(cls, 1), :, :] = jnp.where(mask, zero_w, row)[None]
        hit = cbm & mask
        updc = hit & ((best < -0.5) | (cls < bcls))
        bcls = jnp.where(updc, cls, bcls)
        best = jnp.where(hit, 0.0, best)
        cbx_ref[0] = jnp.where(updc, x1, cbx_ref[0])
        cbx_ref[1] = jnp.where(updc, y1, cbx_ref[1])
        cbx_ref[2] = jnp.where(updc, x2, cbx_ref[2])
        cbx_ref[3] = jnp.where(updc, y2, cbx_ref[3])
        best = jnp.where(selm, -1.0, best)
        bcls = jnp.where(selm, C, bcls)
        best_ref[...] = best
        bcls_ref[...] = bcls
        comm_ref[...] = jnp.where(selm, 1, comm)
        aff = (jnp.logical_not(committed) & jnp.logical_not(selm)
               & mask & (bcls == cls))
        new_row = jnp.where(mask, zero_w, row)
        excl = jnp.where(subf == cls, -jnp.inf, dr_pre)
        cm_excl = jnp.max(excl, axis=0)
        cm = jnp.maximum(cm_excl, new_row)
        ca = jnp.minimum(
            jnp.min(jnp.where(excl == cm, subf, jnp.int32(C)), axis=0),
            jnp.where(new_row == cm, cls, jnp.int32(C)))
        best_ref[...] = jnp.where(aff, cm, best_ref[...])
        bcls_ref[...] = jnp.where(aff, ca, bcls_ref[...])
        cst_ref[...] = jnp.where(aff, 1, cst_ref[...])

        return carry

    jax.lax.fori_loop(0, n_steps, body, 0)


def kernel(x, boxes_per_cls, W, b):
    N, D = x.shape
    C = W.shape[0]
    CP = ((C + 7) // 8) * 8
    NP = 1024
    b2 = b.reshape(1, C)
    logits, dists = pl.pallas_call(
        _dense_kernel,
        out_shape=(
            jax.ShapeDtypeStruct((N, C), jnp.float32),
            jax.ShapeDtypeStruct((C, NP), jnp.float32),
        ),
    )(x, W, b2)
    dists2 = dists.reshape(C, 8, 128)
    boxesT = jnp.transpose(boxes_per_cls, (1, 2, 0))
    boxesP = jnp.concatenate(
        [boxesT, jnp.zeros((C, 4, NP - N), jnp.float32)], axis=2
    ).reshape(C, 4, 8, 128)
    preds = pl.pallas_call(
        functools.partial(_nms_kernel, n_steps=N),
        out_shape=jax.ShapeDtypeStruct((8, 128), jnp.int32),
        scratch_shapes=[
            pltpu.VMEM((CP, 8, 128), jnp.float32),
            pltpu.VMEM((8, 128), jnp.float32),
            pltpu.VMEM((8, 128), jnp.int32),
            pltpu.VMEM((8, 128), jnp.int32),
            pltpu.VMEM((4, 8, 128), jnp.float32),
            pltpu.VMEM((8, 128), jnp.int32),
            pltpu.VMEM((4, 8, 128), jnp.float32),
        ],
    )(dists2, boxesP)
    return logits, preds.reshape(NP)[:N]
